# baseline (speedup 1.0000x reference)
import functools

import jax
import jax.numpy as jnp
from jax import lax
from jax.experimental import pallas as pl
from jax.experimental.pallas import tpu as pltpu
from jax.experimental.pallas import tpu_sc as plsc

L = 16
POOL = 16
HW = 512
PR = HW // POOL
NBLK = PR * PR
NCHUNK = NBLK // L
CPR = PR // L
OUT_LEN = NBLK + 1
ROW_PAD = ((OUT_LEN + L - 1) // L) * L
B = 256
OUT_ROWS_PER_TILE = B // 32


def _tc_pool_body(x_ref, m_ref):
    x = x_ref[0, 0]
    b = (x > 0.0).astype(jnp.float32)
    r1 = lax.broadcasted_iota(jnp.int32, (PR, HW), 0)
    c1 = lax.broadcasted_iota(jnp.int32, (PR, HW), 1)
    p_left = (c1 // POOL == r1).astype(jnp.float32)
    r2 = lax.broadcasted_iota(jnp.int32, (HW, PR), 0)
    c2 = lax.broadcasted_iota(jnp.int32, (HW, PR), 1)
    p_right = (r2 // POOL == c2).astype(jnp.float32)
    rows = jnp.dot(p_left, b, preferred_element_type=jnp.float32)
    counts = jnp.dot(rows, p_right, preferred_element_type=jnp.float32)
    m_ref[...] = (counts > 0.5).astype(jnp.int32)


_tc_pool = pl.pallas_call(
    _tc_pool_body,
    out_shape=jax.ShapeDtypeStruct((PR, PR), jnp.int32),
    grid=(1,),
    in_specs=[pl.BlockSpec((1, 1, HW, HW), lambda i: (0, 0, 0, 0))],
    out_specs=pl.BlockSpec((PR, PR), lambda i: (0, 0)),
)


_mesh = plsc.VectorSubcoreMesh(core_axis_name="c", subcore_axis_name="s")


@functools.partial(
    pl.kernel,
    out_type=jax.ShapeDtypeStruct((B, OUT_LEN), jnp.int32),
    mesh=_mesh,
    compiler_params=pltpu.CompilerParams(needs_layout_passes=False,
                                         use_tc_tiling_on_sc=False),
    scratch_types=[
        pltpu.VMEM((PR, PR), jnp.int32),
        pltpu.VMEM((ROW_PAD,), jnp.int32),
        pltpu.VMEM((OUT_ROWS_PER_TILE, OUT_LEN), jnp.int32),
        pltpu.VMEM_SHARED((OUT_ROWS_PER_TILE, OUT_LEN), jnp.int32),
        pltpu.SemaphoreType.DMA,
    ],
)
def _sc_compact_broadcast(m_hbm, out_hbm, mv, row_v, rep_v, shared_rep, sem):
    c = lax.axis_index("c")
    s = lax.axis_index("s")
    lanes = lax.broadcasted_iota(jnp.int32, (L,), 0)

    @pl.when(s == 0)
    def _compact():
        pltpu.sync_copy(m_hbm, mv)
        one = jnp.ones((L,), jnp.int32)
        row_v[pl.ds(0, L)] = jnp.where(lanes == 0, 0, one)
        for t in range(1, ROW_PAD // L):
            row_v[pl.ds(t * L, L)] = one
        masks, ranks, counts = [], [], []
        for t in range(NCHUNK):
            m_vec = mv[t // CPR, pl.ds((t % CPR) * L, L)]
            masks.append(m_vec > 0)
            ranks.append(plsc.cumsum(m_vec))
            counts.append(jnp.sum(m_vec))
        carry = jnp.int32(0)
        for t in range(NCHUNK):
            idx = ranks[t] + carry
            vals = lanes + (t * L + 1)
            plsc.store_scatter(row_v, [idx], vals, mask=masks[t])
            carry = carry + counts[t]
        pubs = [
            pltpu.async_copy(row_v.at[pl.ds(0, OUT_LEN)], shared_rep.at[i],
                             sem)
            for i in range(OUT_ROWS_PER_TILE)
        ]
        for cp in pubs:
            cp.wait()

    plsc.subcore_barrier()

    pltpu.sync_copy(shared_rep, rep_v)
    base = (s * 2 + c) * OUT_ROWS_PER_TILE
    pltpu.sync_copy(rep_v, out_hbm.at[pl.ds(base, OUT_ROWS_PER_TILE)])


def kernel(ones_mask):
    return _sc_compact_broadcast(_tc_pool(ones_mask))


# speedup vs baseline: 1.0081x; 1.0081x over previous
import functools

import jax
import jax.numpy as jnp
from jax import lax
from jax.experimental import pallas as pl
from jax.experimental.pallas import tpu as pltpu
from jax.experimental.pallas import tpu_sc as plsc

L = 16
POOL = 16
HW = 512
PR = HW // POOL
NBLK = PR * PR
NCHUNK = NBLK // L
CPR = PR // L
OUT_LEN = NBLK + 1
ROW_PAD = ((OUT_LEN + L - 1) // L) * L
B = 256
OUT_ROWS_PER_TILE = B // 32


def _tc_pool_body(x_ref, m_ref):
    x = x_ref[0, 0]
    b = (x > 0.0).astype(jnp.float32)
    r1 = lax.broadcasted_iota(jnp.int32, (PR, HW), 0)
    c1 = lax.broadcasted_iota(jnp.int32, (PR, HW), 1)
    p_left = (c1 // POOL == r1).astype(jnp.float32)
    r2 = lax.broadcasted_iota(jnp.int32, (HW, PR), 0)
    c2 = lax.broadcasted_iota(jnp.int32, (HW, PR), 1)
    p_right = (r2 // POOL == c2).astype(jnp.float32)
    rows = jnp.dot(p_left, b, preferred_element_type=jnp.float32)
    counts = jnp.dot(rows, p_right, preferred_element_type=jnp.float32)
    m_ref[...] = (counts > 0.5).astype(jnp.int32)


_tc_pool = pl.pallas_call(
    _tc_pool_body,
    out_shape=jax.ShapeDtypeStruct((PR, PR), jnp.int32),
    grid=(1,),
    in_specs=[pl.BlockSpec((1, 1, HW, HW), lambda i: (0, 0, 0, 0))],
    out_specs=pl.BlockSpec((PR, PR), lambda i: (0, 0)),
)


_mesh = plsc.VectorSubcoreMesh(core_axis_name="c", subcore_axis_name="s")


@functools.partial(
    pl.kernel,
    out_type=jax.ShapeDtypeStruct((B, OUT_LEN), jnp.int32),
    mesh=_mesh,
    compiler_params=pltpu.CompilerParams(needs_layout_passes=False,
                                         use_tc_tiling_on_sc=False),
    scratch_types=[
        pltpu.VMEM((PR, PR), jnp.int32),
        pltpu.VMEM((ROW_PAD,), jnp.int32),
        pltpu.VMEM((OUT_ROWS_PER_TILE, OUT_LEN), jnp.int32),
        pltpu.VMEM_SHARED((ROW_PAD,), jnp.int32),
        pltpu.SemaphoreType.DMA,
    ],
)
def _sc_compact_broadcast(m_hbm, out_hbm, mv, row_v, rep_v, shared_row, sem):
    c = lax.axis_index("c")
    s = lax.axis_index("s")
    lanes = lax.broadcasted_iota(jnp.int32, (L,), 0)

    @pl.when(s == 0)
    def _compact():
        pltpu.sync_copy(m_hbm, mv)
        one = jnp.ones((L,), jnp.int32)
        row_v[pl.ds(0, L)] = jnp.where(lanes == 0, 0, one)
        for t in range(1, ROW_PAD // L):
            row_v[pl.ds(t * L, L)] = one
        masks, ranks, counts = [], [], []
        for t in range(NCHUNK):
            m_vec = mv[t // CPR, pl.ds((t % CPR) * L, L)]
            masks.append(m_vec > 0)
            ranks.append(plsc.cumsum(m_vec))
            counts.append(jnp.sum(m_vec))
        carry = jnp.int32(0)
        for t in range(NCHUNK):
            idx = ranks[t] + carry
            vals = lanes + (t * L + 1)
            plsc.store_scatter(row_v, [idx], vals, mask=masks[t])
            carry = carry + counts[t]
        pltpu.sync_copy(row_v, shared_row)

    plsc.subcore_barrier()

    copies = [
        pltpu.async_copy(shared_row.at[pl.ds(0, OUT_LEN)], rep_v.at[i], sem)
        for i in range(OUT_ROWS_PER_TILE)
    ]
    for cp in copies:
        cp.wait()
    base = (s * 2 + c) * OUT_ROWS_PER_TILE
    pltpu.sync_copy(rep_v, out_hbm.at[pl.ds(base, OUT_ROWS_PER_TILE)])


def kernel(ones_mask):
    return _sc_compact_broadcast(_tc_pool(ones_mask))
